# baseline (speedup 1.0000x reference)
import functools
import jax
import jax.numpy as jnp
from jax import lax
from jax.experimental import pallas as pl
from jax.experimental.pallas import tpu as pltpu
from jax.experimental.pallas import tpu_sc as plsc

N = 10000
E = 320000
D = 128
H = 128

_NC = 2
_NS = 16
_NW = _NC * _NS


def _k1_body(x_ref, wl_ref, bl_ref, w1_ref, p_ref, q_ref):
    h = jnp.dot(x_ref[...], wl_ref[...], preferred_element_type=jnp.float32)
    h = h + bl_ref[...]
    wa = w1_ref[:D, :] - w1_ref[D:, :]
    wb = w1_ref[D:, :]
    p_ref[...] = jnp.dot(h, wa, preferred_element_type=jnp.float32)
    q_ref[...] = jnp.dot(h, wb, preferred_element_type=jnp.float32)


def _node_matmuls(x, W_lin1, b_lin1, W1):
    blk = 1000
    grid = (N // blk,)
    return pl.pallas_call(
        _k1_body,
        grid=grid,
        in_specs=[
            pl.BlockSpec((blk, D), lambda i: (i, 0)),
            pl.BlockSpec((D, D), lambda i: (0, 0)),
            pl.BlockSpec((1, D), lambda i: (0, 0)),
            pl.BlockSpec((2 * D, H), lambda i: (0, 0)),
        ],
        out_specs=[
            pl.BlockSpec((blk, H), lambda i: (i, 0)),
            pl.BlockSpec((blk, H), lambda i: (i, 0)),
        ],
        out_shape=[
            jax.ShapeDtypeStruct((N, H), jnp.float32),
            jax.ShapeDtypeStruct((N, H), jnp.float32),
        ],
    )(x, W_lin1, b_lin1.reshape(1, D), W1)


_EPW = E // _NW
_CH = 200


def _k2_body(dst_hbm, src_hbm, p_hbm, q_hbm, z_hbm,
             idxd_v, idxs_v, bufp, bufq, semp, semq):
    wid = lax.axis_index("s") * _NC + lax.axis_index("c")
    base = wid * _EPW
    nch = _EPW // _CH

    def idx_and_gather(i):
        ho = pl.multiple_of((i % 2) * _CH, 8)
        off = base + i * _CH
        pltpu.sync_copy(dst_hbm.at[pl.ds(off, _CH)], idxd_v.at[pl.ds(ho, _CH)])
        pltpu.sync_copy(src_hbm.at[pl.ds(off, _CH)], idxs_v.at[pl.ds(ho, _CH)])
        pltpu.async_copy(p_hbm.at[idxd_v.at[pl.ds(ho, _CH)]],
                         bufp.at[pl.ds(ho, _CH)], semp)
        pltpu.async_copy(q_hbm.at[idxs_v.at[pl.ds(ho, _CH)]],
                         bufq.at[pl.ds(ho, _CH)], semq)

    idx_and_gather(0)

    def chunk(i, carry):
        ho = pl.multiple_of((i % 2) * _CH, 8)
        off = base + i * _CH
        pltpu.make_async_copy(p_hbm.at[idxd_v.at[pl.ds(ho, _CH)]],
                              bufp.at[pl.ds(ho, _CH)], semp).wait()
        pltpu.make_async_copy(q_hbm.at[idxs_v.at[pl.ds(ho, _CH)]],
                              bufq.at[pl.ds(ho, _CH)], semq).wait()

        @pl.when(i < nch - 1)
        def _():
            idx_and_gather(i + 1)

        def row(r, c2):
            for c in range(H // 16):
                sl = pl.ds(c * 16, 16)
                bufp[ho + r, sl] = bufp[ho + r, sl] + bufq[ho + r, sl]
            return c2

        lax.fori_loop(0, _CH, row, 0)
        pltpu.sync_copy(bufp.at[pl.ds(ho, _CH)], z_hbm.at[pl.ds(off, _CH)])
        return carry

    lax.fori_loop(0, nch, chunk, 0)


def _edge_gather(dst, src, p, q):
    mesh = plsc.VectorSubcoreMesh(core_axis_name="c", subcore_axis_name="s")
    f = functools.partial(
        pl.kernel,
        out_type=jax.ShapeDtypeStruct((E, H), jnp.float32),
        mesh=mesh,
        compiler_params=pltpu.CompilerParams(needs_layout_passes=False),
        scratch_types=[
            pltpu.VMEM((2 * _CH,), jnp.int32),
            pltpu.VMEM((2 * _CH,), jnp.int32),
            pltpu.VMEM((2 * _CH, H), jnp.float32),
            pltpu.VMEM((2 * _CH, H), jnp.float32),
            pltpu.SemaphoreType.DMA,
            pltpu.SemaphoreType.DMA,
        ],
    )(_k2_body)
    return f(dst, src, p, q)


def _k3_body(z_ref, b1_ref, w2_ref, b2_ref, y_ref):
    z = jnp.maximum(z_ref[...] + b1_ref[...], 0.0)
    y = jnp.dot(z, w2_ref[...], preferred_element_type=jnp.float32)
    y_ref[...] = jnp.maximum(y + b2_ref[...], 0.0)


def _edge_mlp(z, b1, W2, b2):
    blk = 2000
    grid = (E // blk,)
    return pl.pallas_call(
        _k3_body,
        grid=grid,
        in_specs=[
            pl.BlockSpec((blk, H), lambda i: (i, 0)),
            pl.BlockSpec((1, H), lambda i: (0, 0)),
            pl.BlockSpec((H, H), lambda i: (0, 0)),
            pl.BlockSpec((1, H), lambda i: (0, 0)),
        ],
        out_specs=pl.BlockSpec((blk, H), lambda i: (i, 0)),
        out_shape=jax.ShapeDtypeStruct((E, H), jnp.float32),
    )(z, b1.reshape(1, H), W2, b2.reshape(1, H))


_G = 632
_NPAD = 16 * _G
_CH2 = 2000
_GCH = 320
_ESH = E // 2


def _k4_body(dst_hbm, y_hbm, out_hbm, dwin, pid, pd, rows, acc, semg, semg2):
    g = lax.axis_index("s")
    r = lax.axis_index("c")
    lo = g * _G
    ebase = r * _ESH
    iota = lax.iota(jnp.int32, 16)
    zeros16 = jnp.zeros((16,), jnp.float32)

    def zrow(row, c2):
        for c in range(H // 16):
            acc[row, pl.ds(c * 16, 16)] = zeros16
        return c2

    lax.fori_loop(0, _G, zrow, 0)

    def zpid(i, c2):
        pid[pl.ds(i * 16, 16)] = jnp.zeros((16,), jnp.int32)
        return c2

    lax.fori_loop(0, _GCH // 16, zpid, 0)

    def rmw_range(jlo, jhi):
        def rmw(j, c4):
            jvec = jnp.full((16,), j, jnp.int32)
            dvec = plsc.load_gather(pd, [jvec]) - lo
            for c in range(H // 16):
                colv = jnp.full((16,), c * 16, jnp.int32) + iota
                cur = plsc.load_gather(acc, [dvec, colv])
                yv = rows[j, pl.ds(c * 16, 16)]
                plsc.store_scatter(acc, [dvec, colv], jnp.maximum(cur, yv))
            return c4

        lax.fori_loop(jlo, jhi, rmw, 0)

    _HG = _GCH // 2

    def flush(cnt):
        cp0 = pltpu.async_copy(y_hbm.at[pid.at[pl.ds(0, _HG)]],
                               rows.at[pl.ds(0, _HG)], semg)
        cp1 = pltpu.async_copy(y_hbm.at[pid.at[pl.ds(_HG, _HG)]],
                               rows.at[pl.ds(_HG, _HG)], semg2)
        cp0.wait()
        rmw_range(0, jnp.minimum(cnt, _HG))
        cp1.wait()
        rmw_range(_HG, jnp.maximum(cnt, _HG))

    def chunk(ci, off):
        cb = ebase + ci * _CH2
        pltpu.sync_copy(dst_hbm.at[pl.ds(cb, _CH2)], dwin)

        def vb(v, off):
            d = dwin[pl.ds(v * 16, 16)]
            mask = (d >= lo) & (d < lo + _G)
            mi = jnp.where(mask, 1, 0)
            pos = plsc.cumsum(mi) - 1 + off
            eid = cb + v * 16 + iota
            plsc.store_scatter(pid, [pos], eid, mask=mask)
            plsc.store_scatter(pd, [pos], d, mask=mask)
            off = off + plsc.all_reduce_population_count(mask)[0]

            def do_flush(o):
                flush(o)
                return jnp.int32(0)

            return lax.cond(off >= _GCH - 16, do_flush, lambda o: o, off)

        return lax.fori_loop(0, _CH2 // 16, vb, off, unroll=5)

    off = lax.fori_loop(0, _ESH // _CH2, chunk, jnp.int32(0))
    flush(off)
    pltpu.sync_copy(acc, out_hbm.at[r, pl.ds(lo, _G)])


def _scatter_max(dst, y):
    mesh = plsc.VectorSubcoreMesh(core_axis_name="c", subcore_axis_name="s")
    f = functools.partial(
        pl.kernel,
        out_type=jax.ShapeDtypeStruct((2, _NPAD, H), jnp.float32),
        mesh=mesh,
        compiler_params=pltpu.CompilerParams(needs_layout_passes=False),
        scratch_types=[
            pltpu.VMEM((_CH2,), jnp.int32),
            pltpu.VMEM((_GCH,), jnp.int32),
            pltpu.VMEM((_GCH,), jnp.int32),
            pltpu.VMEM((_GCH, H), jnp.float32),
            pltpu.VMEM((_G, H), jnp.float32),
            pltpu.SemaphoreType.DMA,
            pltpu.SemaphoreType.DMA,
        ],
    )(_k4_body)
    return f(dst, y)


def _k5_body(a_ref, b_ref, o_ref):
    o_ref[...] = jnp.maximum(a_ref[0], b_ref[0])


def _shard_merge(agg2):
    blk = 632
    grid = (_NPAD // blk,)
    return pl.pallas_call(
        _k5_body,
        grid=grid,
        in_specs=[
            pl.BlockSpec((1, blk, H), lambda i: (0, i, 0)),
            pl.BlockSpec((1, blk, H), lambda i: (1, i, 0)),
        ],
        out_specs=pl.BlockSpec((blk, H), lambda i: (i, 0)),
        out_shape=jax.ShapeDtypeStruct((_NPAD, H), jnp.float32),
    )(agg2, agg2)


def kernel(x, edge_index, W_lin1, b_lin1, W1, b1, W2, b2):
    src = edge_index[0]
    dst = edge_index[1]
    p, q = _node_matmuls(x, W_lin1, b_lin1, W1)
    z = _edge_gather(dst, src, p, q)
    y = _edge_mlp(z, b1, W2, b2)
    agg2 = _scatter_max(dst, y)
    agg = _shard_merge(agg2)
    return agg[:N]


# speedup vs baseline: 1.1969x; 1.1969x over previous
import functools
import jax
import jax.numpy as jnp
from jax import lax
from jax.experimental import pallas as pl
from jax.experimental.pallas import tpu as pltpu
from jax.experimental.pallas import tpu_sc as plsc

N = 10000
E = 320000
D = 128
H = 128

_NC = 2
_NS = 16
_NW = _NC * _NS


def _k1_body(x_ref, wl_ref, bl_ref, w1_ref, p_ref, q_ref):
    h = jnp.dot(x_ref[...], wl_ref[...], preferred_element_type=jnp.float32)
    h = h + bl_ref[...]
    wa = w1_ref[:D, :] - w1_ref[D:, :]
    wb = w1_ref[D:, :]
    p_ref[...] = jnp.dot(h, wa, preferred_element_type=jnp.float32)
    q_ref[...] = jnp.dot(h, wb, preferred_element_type=jnp.float32)


def _node_matmuls(x, W_lin1, b_lin1, W1):
    blk = 1000
    grid = (N // blk,)
    return pl.pallas_call(
        _k1_body,
        grid=grid,
        in_specs=[
            pl.BlockSpec((blk, D), lambda i: (i, 0)),
            pl.BlockSpec((D, D), lambda i: (0, 0)),
            pl.BlockSpec((1, D), lambda i: (0, 0)),
            pl.BlockSpec((2 * D, H), lambda i: (0, 0)),
        ],
        out_specs=[
            pl.BlockSpec((blk, H), lambda i: (i, 0)),
            pl.BlockSpec((blk, H), lambda i: (i, 0)),
        ],
        out_shape=[
            jax.ShapeDtypeStruct((N, H), jnp.float32),
            jax.ShapeDtypeStruct((N, H), jnp.float32),
        ],
    )(x, W_lin1, b_lin1.reshape(1, D), W1)


_EPW = E // _NW
_CH = 400


def _k2_body(dst_hbm, src_hbm, p_hbm, q_hbm, z_hbm,
             idxd_v, idxs_v, bufp, bufq, semp, semq):
    wid = lax.axis_index("s") * _NC + lax.axis_index("c")
    base = wid * _EPW

    def chunk(i, carry):
        off = base + i * _CH
        pltpu.sync_copy(dst_hbm.at[pl.ds(off, _CH)], idxd_v)
        pltpu.sync_copy(src_hbm.at[pl.ds(off, _CH)], idxs_v)
        cp = pltpu.async_copy(p_hbm.at[idxd_v], bufp, semp)
        cq = pltpu.async_copy(q_hbm.at[idxs_v], bufq, semq)
        cp.wait()
        cq.wait()

        def row(r, c2):
            for c in range(H // 16):
                s = pl.ds(c * 16, 16)
                bufp[r, s] = bufp[r, s] + bufq[r, s]
            return c2

        lax.fori_loop(0, _CH, row, 0)
        pltpu.sync_copy(bufp, z_hbm.at[pl.ds(off, _CH)])
        return carry

    lax.fori_loop(0, _EPW // _CH, chunk, 0)


def _edge_gather(dst, src, p, q):
    mesh = plsc.VectorSubcoreMesh(core_axis_name="c", subcore_axis_name="s")
    f = functools.partial(
        pl.kernel,
        out_type=jax.ShapeDtypeStruct((E, H), jnp.float32),
        mesh=mesh,
        compiler_params=pltpu.CompilerParams(needs_layout_passes=False),
        scratch_types=[
            pltpu.VMEM((_CH,), jnp.int32),
            pltpu.VMEM((_CH,), jnp.int32),
            pltpu.VMEM((_CH, H), jnp.float32),
            pltpu.VMEM((_CH, H), jnp.float32),
            pltpu.SemaphoreType.DMA,
            pltpu.SemaphoreType.DMA,
        ],
    )(_k2_body)
    return f(dst, src, p, q)


def _k3_body(z_ref, b1_ref, w2_ref, b2_ref, y_ref):
    z = jnp.maximum(z_ref[...] + b1_ref[...], 0.0)
    y = jnp.dot(z, w2_ref[...], preferred_element_type=jnp.float32)
    y_ref[...] = jnp.maximum(y + b2_ref[...], 0.0)


def _edge_mlp(z, b1, W2, b2):
    blk = 2000
    grid = (E // blk,)
    return pl.pallas_call(
        _k3_body,
        grid=grid,
        in_specs=[
            pl.BlockSpec((blk, H), lambda i: (i, 0)),
            pl.BlockSpec((1, H), lambda i: (0, 0)),
            pl.BlockSpec((H, H), lambda i: (0, 0)),
            pl.BlockSpec((1, H), lambda i: (0, 0)),
        ],
        out_specs=pl.BlockSpec((blk, H), lambda i: (i, 0)),
        out_shape=jax.ShapeDtypeStruct((E, H), jnp.float32),
    )(z, b1.reshape(1, H), W2, b2.reshape(1, H))


_G = 632
_NPAD = 16 * _G
_CH2 = 2000
_GCH = 320
_ESH = E // 2


def _k4_body(dst_hbm, y_hbm, out_hbm, dwin, pid, pd, rows, acc, semg, semg2):
    g = lax.axis_index("s")
    r = lax.axis_index("c")
    lo = g * _G
    ebase = r * _ESH
    iota = lax.iota(jnp.int32, 16)
    zeros16 = jnp.zeros((16,), jnp.float32)

    def zrow(row, c2):
        for c in range(H // 16):
            acc[row, pl.ds(c * 16, 16)] = zeros16
        return c2

    lax.fori_loop(0, _G, zrow, 0)

    def zpid(i, c2):
        pid[pl.ds(i * 16, 16)] = jnp.zeros((16,), jnp.int32)
        return c2

    lax.fori_loop(0, _GCH // 16, zpid, 0)

    def rmw_range(jlo, jhi):
        def rmw(j, c4):
            jvec = jnp.full((16,), j, jnp.int32)
            dvec = plsc.load_gather(pd, [jvec]) - lo
            for c in range(H // 16):
                colv = jnp.full((16,), c * 16, jnp.int32) + iota
                cur = plsc.load_gather(acc, [dvec, colv])
                yv = rows[j, pl.ds(c * 16, 16)]
                plsc.store_scatter(acc, [dvec, colv], jnp.maximum(cur, yv))
            return c4

        lax.fori_loop(jlo, jhi, rmw, 0)

    _HG = _GCH // 2

    def flush(cnt):
        cp0 = pltpu.async_copy(y_hbm.at[pid.at[pl.ds(0, _HG)]],
                               rows.at[pl.ds(0, _HG)], semg)
        cp1 = pltpu.async_copy(y_hbm.at[pid.at[pl.ds(_HG, _HG)]],
                               rows.at[pl.ds(_HG, _HG)], semg2)
        cp0.wait()
        rmw_range(0, jnp.minimum(cnt, _HG))
        cp1.wait()
        rmw_range(_HG, jnp.maximum(cnt, _HG))

    def chunk(ci, off):
        cb = ebase + ci * _CH2
        pltpu.sync_copy(dst_hbm.at[pl.ds(cb, _CH2)], dwin)

        def vb(v, off):
            d = dwin[pl.ds(v * 16, 16)]
            mask = (d >= lo) & (d < lo + _G)
            mi = jnp.where(mask, 1, 0)
            pos = plsc.cumsum(mi) - 1 + off
            eid = cb + v * 16 + iota
            plsc.store_scatter(pid, [pos], eid, mask=mask)
            plsc.store_scatter(pd, [pos], d, mask=mask)
            off = off + plsc.all_reduce_population_count(mask)[0]

            def do_flush(o):
                flush(o)
                return jnp.int32(0)

            return lax.cond(off >= _GCH - 16, do_flush, lambda o: o, off)

        return lax.fori_loop(0, _CH2 // 16, vb, off, unroll=5)

    off = lax.fori_loop(0, _ESH // _CH2, chunk, jnp.int32(0))
    flush(off)
    pltpu.sync_copy(acc, out_hbm.at[r, pl.ds(lo, _G)])


def _scatter_max(dst, y):
    mesh = plsc.VectorSubcoreMesh(core_axis_name="c", subcore_axis_name="s")
    f = functools.partial(
        pl.kernel,
        out_type=jax.ShapeDtypeStruct((2, _NPAD, H), jnp.float32),
        mesh=mesh,
        compiler_params=pltpu.CompilerParams(needs_layout_passes=False),
        scratch_types=[
            pltpu.VMEM((_CH2,), jnp.int32),
            pltpu.VMEM((_GCH,), jnp.int32),
            pltpu.VMEM((_GCH,), jnp.int32),
            pltpu.VMEM((_GCH, H), jnp.float32),
            pltpu.VMEM((_G, H), jnp.float32),
            pltpu.SemaphoreType.DMA,
            pltpu.SemaphoreType.DMA,
        ],
    )(_k4_body)
    return f(dst, y)


def _k5_body(a_ref, b_ref, o_ref):
    o_ref[...] = jnp.maximum(a_ref[0], b_ref[0])


def _shard_merge(agg2):
    blk = 632
    grid = (_NPAD // blk,)
    return pl.pallas_call(
        _k5_body,
        grid=grid,
        in_specs=[
            pl.BlockSpec((1, blk, H), lambda i: (0, i, 0)),
            pl.BlockSpec((1, blk, H), lambda i: (1, i, 0)),
        ],
        out_specs=pl.BlockSpec((blk, H), lambda i: (i, 0)),
        out_shape=jax.ShapeDtypeStruct((_NPAD, H), jnp.float32),
    )(agg2, agg2)


def kernel(x, edge_index, W_lin1, b_lin1, W1, b1, W2, b2):
    src = edge_index[0]
    dst = edge_index[1]
    p, q = _node_matmuls(x, W_lin1, b_lin1, W1)
    z = _edge_gather(dst, src, p, q)
    y = _edge_mlp(z, b1, W2, b2)
    agg2 = _scatter_max(dst, y)
    agg = _shard_merge(agg2)
    return agg[:N]
